# baseline (speedup 1.0000x reference)
import functools

import jax
import jax.numpy as jnp
from jax import lax
from jax.experimental import pallas as pl
from jax.experimental.pallas import tpu as pltpu
from jax.experimental.pallas import tpu_sc as plsc

N_ATOMS = 10000
MAX_B = 32
E = 320000
H = 128
NC = 2
NS = 16
NW = NC * NS
NLANE = 16
NSL = H // NLANE

APW = 320
N_PAD = NW * APW
CA = 4
NCH_A = APW // CA
NB_A = 4
EPW = 10240
E_PAD = NW * EPW
CE = 40
NCH_E = EPW // CE
NB_E = 4

_mesh = plsc.VectorSubcoreMesh(core_axis_name="c", subcore_axis_name="s")


def _worker_id():
    return lax.axis_index("s") * NC + lax.axis_index("c")


@functools.partial(
    pl.kernel,
    out_type=jax.ShapeDtypeStruct((N_PAD, H), jnp.float32),
    mesh=_mesh,
    scratch_types=(
        [pltpu.VMEM((NCH_A, 1, CA * MAX_B), jnp.int32)]
        + [pltpu.VMEM((CA * MAX_B, H), jnp.float32) for _ in range(NB_A)]
        + [pltpu.VMEM((CA, H), jnp.float32) for _ in range(NB_A)]
        + [pltpu.SemaphoreType.DMA for _ in range(2 * NB_A)]
    ),
)
def _segsum_relu(msg_hbm, a2b_hbm, out_hbm, idx_v, *bufs):
    rows = bufs[:NB_A]
    accs = bufs[NB_A:2 * NB_A]
    sg = bufs[2 * NB_A:3 * NB_A]
    sw = bufs[3 * NB_A:4 * NB_A]
    wid = _worker_id()

    def gather(b, c, fire):
        src = msg_hbm.at[idx_v.at[c, 0]]
        if fire:
            return pltpu.async_copy(src, rows[b], sg[b])
        return pltpu.make_async_copy(src, rows[b], sg[b])

    def wb(b, c, fire):
        dst = out_hbm.at[pl.ds(wid * APW + c * CA, CA)]
        if fire:
            return pltpu.async_copy(accs[b], dst, sw[b])
        return pltpu.make_async_copy(accs[b], dst, sw[b])

    pltpu.sync_copy(a2b_hbm.at[pl.ds(wid * NCH_A, NCH_A)], idx_v)
    for b in range(NB_A):
        gather(b, b, True)

    def outer(o, carry):
        for b in range(NB_A):
            c = o * NB_A + b

            @pl.when(c >= NB_A)
            def _():
                wb(b, c - NB_A, False).wait()

            gather(b, c, False).wait()

            for a in range(CA):
                def nb_body(k, ac):
                    r = a * MAX_B + k
                    return tuple(
                        ac[s] + jnp.maximum(
                            rows[b][r, pl.ds(s * NLANE, NLANE)], 0.0)
                        for s in range(NSL))

                zero = tuple(jnp.zeros((NLANE,), jnp.float32)
                             for _ in range(NSL))
                acv = lax.fori_loop(0, MAX_B, nb_body, zero)
                for s in range(NSL):
                    accs[b][a, pl.ds(s * NLANE, NLANE)] = acv[s]

            wb(b, c, True)

            @pl.when(c + NB_A < NCH_A)
            def _():
                gather(b, c + NB_A, True)
        return carry

    lax.fori_loop(0, NCH_A // NB_A, outer, 0)
    for b in range(NB_A):
        wb(b, NCH_A - NB_A + b, False).wait()


@functools.partial(
    pl.kernel,
    out_type=jax.ShapeDtypeStruct((E_PAD, H), jnp.float32),
    mesh=_mesh,
    scratch_types=(
        [pltpu.VMEM((NCH_E, 1, CE), jnp.int32),
         pltpu.VMEM((NCH_E, 1, CE), jnp.int32)]
        + [pltpu.VMEM((CE, H), jnp.float32) for _ in range(3 * NB_E)]
        + [pltpu.SemaphoreType.DMA for _ in range(3 * NB_E)]
    ),
)
def _edge_delta(amsg_hbm, msg_hbm, b2a_hbm, b2revb_hbm, t_hbm,
                idxa_v, idxm_v, *bufs):
    arows = bufs[:NB_E]
    mrows = bufs[NB_E:2 * NB_E]
    trows = bufs[2 * NB_E:3 * NB_E]
    sga = bufs[3 * NB_E:4 * NB_E]
    sgm = bufs[4 * NB_E:5 * NB_E]
    swb = bufs[5 * NB_E:6 * NB_E]
    wid = _worker_id()

    def gathers(b, c, fire):
        mk = pltpu.async_copy if fire else pltpu.make_async_copy
        ca = mk(amsg_hbm.at[idxa_v.at[c, 0]], arows[b], sga[b])
        cm = mk(msg_hbm.at[idxm_v.at[c, 0]], mrows[b], sgm[b])
        return ca, cm

    def wb(b, c, fire):
        mk = pltpu.async_copy if fire else pltpu.make_async_copy
        return mk(trows[b], t_hbm.at[pl.ds(wid * EPW + c * CE, CE)], swb[b])

    pltpu.sync_copy(b2a_hbm.at[pl.ds(wid * NCH_E, NCH_E)], idxa_v)
    pltpu.sync_copy(b2revb_hbm.at[pl.ds(wid * NCH_E, NCH_E)], idxm_v)
    for b in range(NB_E):
        gathers(b, b, True)

    def outer(o, carry):
        for b in range(NB_E):
            c = o * NB_E + b

            @pl.when(c >= NB_E)
            def _():
                wb(b, c - NB_E, False).wait()

            ca, cm = gathers(b, c, False)
            ca.wait()
            cm.wait()

            def edge_body(e, carry2):
                for s in range(NSL):
                    sl = pl.ds(s * NLANE, NLANE)
                    trows[b][e, sl] = arows[b][e, sl] - jnp.maximum(
                        mrows[b][e, sl], 0.0)
                return carry2

            lax.fori_loop(0, CE, edge_body, 0)
            wb(b, c, True)

            @pl.when(c + NB_E < NCH_E)
            def _():
                gathers(b, c + NB_E, True)
        return carry

    lax.fori_loop(0, NCH_E // NB_E, outer, 0)
    for b in range(NB_E):
        wb(b, NCH_E - NB_E + b, False).wait()


def _mm_in(f_bonds, W_i):
    BE = 1000

    def body(fb_ref, wi_ref, out_ref):
        out_ref[...] = jnp.dot(fb_ref[...], wi_ref[...],
                               preferred_element_type=jnp.float32)

    return pl.pallas_call(
        body,
        grid=(E // BE,),
        in_specs=[
            pl.BlockSpec((BE, f_bonds.shape[1]), lambda i: (i, 0)),
            pl.BlockSpec(W_i.shape, lambda i: (0, 0)),
        ],
        out_specs=pl.BlockSpec((BE, H), lambda i: (i, 0)),
        out_shape=jax.ShapeDtypeStruct((E, H), jnp.float32),
    )(f_bonds, W_i)


def _mm_update(inp, t, W_h):
    BE = 1000

    def body(inp_ref, t_ref, wh_ref, out_ref):
        out_ref[...] = inp_ref[...] + jnp.dot(
            t_ref[...], wh_ref[...], preferred_element_type=jnp.float32)

    return pl.pallas_call(
        body,
        grid=(E // BE,),
        in_specs=[
            pl.BlockSpec((BE, H), lambda i: (i, 0)),
            pl.BlockSpec((BE, H), lambda i: (i, 0)),
            pl.BlockSpec((H, H), lambda i: (0, 0)),
        ],
        out_specs=pl.BlockSpec((BE, H), lambda i: (i, 0)),
        out_shape=jax.ShapeDtypeStruct((E, H), jnp.float32),
    )(inp, t, W_h)


def _readout(f_atoms, a_message, W_o, b_o, W1, b1, W2, b2, W3, b3):
    BA = 2000
    Wo_a = W_o[:H]
    Wo_m = W_o[H:]
    W3p = jnp.zeros((H, H), jnp.float32).at[:, :W3.shape[1]].set(W3)
    b3p = jnp.zeros((1, H), jnp.float32).at[0, :b3.shape[0]].set(b3)

    def body(fa_ref, am_ref, woa_ref, wom_ref, bo_ref, w1_ref, b1_ref,
             w2_ref, b2_ref, w3_ref, b3_ref, out_ref):
        ah = jax.nn.relu(
            jnp.dot(fa_ref[...], woa_ref[...],
                    preferred_element_type=jnp.float32)
            + jnp.dot(am_ref[...], wom_ref[...],
                      preferred_element_type=jnp.float32)
            + bo_ref[...])
        h = jax.nn.relu(jnp.dot(ah, w1_ref[...],
                                preferred_element_type=jnp.float32)
                        + b1_ref[...])
        h = jax.nn.relu(jnp.dot(h, w2_ref[...],
                                preferred_element_type=jnp.float32)
                        + b2_ref[...])
        out_ref[...] = jax.nn.sigmoid(
            jnp.dot(h, w3_ref[...], preferred_element_type=jnp.float32)
            + b3_ref[...])

    full = lambda shape: pl.BlockSpec(shape, lambda i: (0, 0))
    out = pl.pallas_call(
        body,
        grid=(N_ATOMS // BA,),
        in_specs=[
            pl.BlockSpec((BA, H), lambda i: (i, 0)),
            pl.BlockSpec((BA, H), lambda i: (i, 0)),
            full((H, H)), full((H, H)), full((1, H)),
            full((H, H)), full((1, H)),
            full((H, H)), full((1, H)),
            full((H, H)), full((1, H)),
        ],
        out_specs=pl.BlockSpec((BA, H), lambda i: (i, 0)),
        out_shape=jax.ShapeDtypeStruct((N_ATOMS, H), jnp.float32),
    )(f_atoms, a_message, Wo_a, Wo_m, b_o.reshape(1, H),
      W1, b1.reshape(1, H), W2, b2.reshape(1, H), W3p, b3p)
    return out


def kernel(f_atoms, f_bonds, a2b, b2a, b2revb, W_i, W_h, W_o, b_o,
           W1, b1, W2, b2, W3, b3):
    a2b_2d = jnp.zeros((N_PAD, MAX_B), jnp.int32).at[:N_ATOMS].set(
        a2b).reshape(NW * NCH_A, 1, CA * MAX_B)
    b2a_2d = jnp.zeros((E_PAD,), jnp.int32).at[:E].set(b2a).reshape(
        NW * NCH_E, 1, CE)
    b2revb_2d = jnp.zeros((E_PAD,), jnp.int32).at[:E].set(b2revb).reshape(
        NW * NCH_E, 1, CE)

    inp = _mm_in(f_bonds, W_i)
    msg_pre = inp
    for _ in range(2):
        a_msg = _segsum_relu(msg_pre, a2b_2d)
        t = _edge_delta(a_msg, msg_pre, b2a_2d, b2revb_2d)
        msg_pre = _mm_update(inp, t, W_h)
    a_message = _segsum_relu(msg_pre, a2b_2d)[:N_ATOMS]
    out = _readout(f_atoms, a_message, W_o, b_o, W1, b1, W2, b2, W3, b3)
    return out[1:, :1]
